# baseline (speedup 1.0000x reference)
import functools
import math

import jax
import jax.numpy as jnp
from jax import lax
from jax.experimental import pallas as pl
from jax.experimental.pallas import tpu as pltpu
from jax.experimental.pallas import tpu_sc as plsc

VOCAB_SIZE = 1000000
D = 64
DP = 128
SCALE = math.sqrt(D)
NC, NS = 2, 16
NW = NC * NS
B = 4096 * 200
PER_W = B // NW
C = 128
NCH = PER_W // C
L = 16


def _make_sc_kernel():
    mesh = plsc.VectorSubcoreMesh(core_axis_name="c", subcore_axis_name="s")

    @functools.partial(
        pl.kernel,
        out_type=jax.ShapeDtypeStruct((B // 2, DP), jnp.float32),
        mesh=mesh,
        compiler_params=pltpu.CompilerParams(use_tc_tiling_on_sc=False,
                                             needs_layout_passes=False),
        scratch_types=[
            pltpu.VMEM((PER_W,), jnp.int32),
            pltpu.VMEM((2, C), jnp.int32),
            pltpu.VMEM((2, C, DP), jnp.float32),
            pltpu.VMEM((2, C // 2, DP), jnp.float32),
            pltpu.SemaphoreType.DMA,
            pltpu.SemaphoreType.DMA,
        ],
    )
    def emb(x_hbm, tab_hbm, out_hbm, idx_v, pidx_v, buf_v, cmp_v, sem0, sem1):
        wid = lax.axis_index("s") * NC + lax.axis_index("c")
        pltpu.sync_copy(x_hbm.at[pl.ds(wid * PER_W, PER_W)], idx_v)
        sems = (sem0, sem1)
        iota = lax.iota(jnp.int32, L)

        def issue(ch, b):
            @pl.loop(0, C // L)
            def _p(g):
                pidx_v[b, pl.ds(g * L, L)] = (
                    idx_v[pl.ds(ch * C + g * L, L)] >> 1)
            pltpu.async_copy(tab_hbm.at[pidx_v.at[b]], buf_v.at[b], sems[b])

        def wait_gather(b):
            pltpu.make_async_copy(
                tab_hbm.at[pidx_v.at[b]], buf_v.at[b], sems[b]).wait()

        for b in range(2):
            issue(b, b)

        @pl.loop(0, NCH, step=2)
        def _chunk(ch):
            for b in range(2):
                cc = ch + b
                wait_gather(b)

                @pl.loop(0, C // L)
                def _g(g):
                    idxv = idx_v[pl.ds(cc * C + g * L, L)]
                    par = (idxv & 1) * D
                    rows = iota + g * L
                    crow = rows >> 1
                    ccol0 = (iota & 1) * D
                    for c in range(D):
                        v = plsc.load_gather(buf_v.at[b], [rows, par + c])
                        plsc.store_scatter(
                            cmp_v.at[b], [crow, ccol0 + c], v * SCALE)

                pltpu.sync_copy(
                    cmp_v.at[b],
                    out_hbm.at[pl.ds((wid * PER_W + cc * C) // 2, C // 2)])

                @pl.when(cc + 2 < NCH)
                def _next():
                    issue(cc + 2, b)

    return emb


_emb = _make_sc_kernel()


def kernel(x, table):
    xf = x.astype(jnp.int32).reshape(-1)
    tab_p = table.reshape(VOCAB_SIZE // 2, DP)
    out2 = _emb(xf, tab_p)
    return out2.reshape(x.shape[0], x.shape[1], D)


# speedup vs baseline: 1.8925x; 1.8925x over previous
import functools
import math

import jax
import jax.numpy as jnp
from jax import lax
from jax.experimental import pallas as pl
from jax.experimental.pallas import tpu as pltpu
from jax.experimental.pallas import tpu_sc as plsc

VOCAB_SIZE = 1000000
D = 64
DP = 128
SCALE = math.sqrt(D)
NC, NS = 2, 16
NW = NC * NS
B = 4096 * 200
PER_W = B // NW
C = 128
NCH = PER_W // C
L = 16


def _make_sc_kernel():
    mesh = plsc.VectorSubcoreMesh(core_axis_name="c", subcore_axis_name="s")

    @functools.partial(
        pl.kernel,
        out_type=jax.ShapeDtypeStruct((B // 2, DP), jnp.float32),
        mesh=mesh,
        compiler_params=pltpu.CompilerParams(use_tc_tiling_on_sc=True),
        scratch_types=[
            pltpu.VMEM((PER_W,), jnp.int32),
            pltpu.VMEM((C,), jnp.int32),
            pltpu.VMEM((C,), jnp.int32),
            pltpu.VMEM((C, DP), jnp.float32),
            pltpu.VMEM((C, DP), jnp.float32),
            pltpu.VMEM((C // 2, DP), jnp.float32),
            pltpu.VMEM((C // 2, DP), jnp.float32),
            pltpu.SemaphoreType.DMA,
            pltpu.SemaphoreType.DMA,
        ],
    )
    def emb(x_hbm, tab_hbm, out_hbm, idx_v, pidx0, pidx1, buf0, buf1,
            cmp0, cmp1, sem0, sem1):
        wid = lax.axis_index("s") * NC + lax.axis_index("c")
        pltpu.sync_copy(x_hbm.at[pl.ds(wid * PER_W, PER_W)], idx_v)
        pidxs = (pidx0, pidx1)
        bufs = (buf0, buf1)
        cmps = (cmp0, cmp1)
        sems = (sem0, sem1)
        orow0 = wid * (PER_W // 2)

        def issue(ch, b):
            @pl.loop(0, C // L)
            def _p(g):
                pidxs[b][pl.ds(g * L, L)] = (
                    idx_v[pl.ds(ch * C + g * L, L)] >> 1)
            pltpu.async_copy(tab_hbm.at[pidxs[b]], bufs[b], sems[b])

        def wait_gather(b):
            pltpu.make_async_copy(
                tab_hbm.at[pidxs[b]], bufs[b], sems[b]).wait()

        for b in range(2):
            issue(b, b)

        @pl.loop(0, NCH, step=2)
        def _chunk(ch):
            for b in range(2):
                cc = ch + b
                wait_gather(b)
                buf, cmp = bufs[b], cmps[b]

                @pl.loop(0, C // L)
                def _g(g):
                    parv = (idx_v[pl.ds(cc * C + g * L, L)] & 1) * D
                    for jj in range(L):
                        off = parv[jj]
                        row = g * L + jj
                        for c in range(0, D, L):
                            v = buf[row, pl.ds(off + c, L)]
                            cmp[g * (L // 2) + jj // 2,
                                pl.ds((jj & 1) * D + c, L)] = v * SCALE

                pltpu.sync_copy(
                    cmp, out_hbm.at[pl.ds(orow0 + cc * (C // 2), C // 2)])

                @pl.when(cc + 2 < NCH)
                def _next():
                    issue(cc + 2, b)

    return emb


_emb = _make_sc_kernel()


def kernel(x, table):
    xf = x.astype(jnp.int32).reshape(-1)
    tab_p = table.reshape(VOCAB_SIZE // 2, DP)
    out2 = _emb(xf, tab_p)
    return out2.reshape(x.shape[0], x.shape[1], D)


# speedup vs baseline: 3.0589x; 1.6163x over previous
import functools
import math

import jax
import jax.numpy as jnp
from jax import lax
from jax.experimental import pallas as pl
from jax.experimental.pallas import tpu as pltpu
from jax.experimental.pallas import tpu_sc as plsc

VOCAB_SIZE = 1000000
D = 64
DP = 128
SCALE = math.sqrt(D)
NC, NS = 2, 16
NW = NC * NS
B = 4096 * 200
PER_W = B // NW
C = 128
NCH = PER_W // C
L = 16


def _make_sc_kernel():
    mesh = plsc.VectorSubcoreMesh(core_axis_name="c", subcore_axis_name="s")

    @functools.partial(
        pl.kernel,
        out_type=jax.ShapeDtypeStruct((B, D), jnp.float32),
        mesh=mesh,
        compiler_params=pltpu.CompilerParams(use_tc_tiling_on_sc=True),
        scratch_types=[
            pltpu.VMEM((PER_W,), jnp.int32),
            pltpu.VMEM((C, DP), jnp.float32),
            pltpu.VMEM((C, DP), jnp.float32),
            pltpu.VMEM((C, D), jnp.float32),
            pltpu.VMEM((C, D), jnp.float32),
            pltpu.SemaphoreType.DMA,
            pltpu.SemaphoreType.DMA,
        ],
    )
    def emb(x_hbm, tab_hbm, out_hbm, idx_v, buf0, buf1, cmp0, cmp1,
            sem0, sem1):
        wid = lax.axis_index("s") * NC + lax.axis_index("c")
        base = wid * PER_W
        pltpu.sync_copy(x_hbm.at[pl.ds(base, PER_W)], idx_v)
        bufs = (buf0, buf1)
        cmps = (cmp0, cmp1)
        sems = (sem0, sem1)

        def issue(ch, b):
            pltpu.async_copy(
                tab_hbm.at[idx_v.at[pl.ds(ch * C, C)]], bufs[b], sems[b])

        def wait_gather(ch, b):
            pltpu.make_async_copy(
                tab_hbm.at[idx_v.at[pl.ds(ch * C, C)]], bufs[b],
                sems[b]).wait()

        for b in range(2):
            issue(b, b)

        @pl.loop(0, NCH, step=2)
        def _chunk(ch):
            for b in range(2):
                cc = ch + b
                wait_gather(cc, b)
                buf, cmp = bufs[b], cmps[b]

                @pl.loop(0, C)
                def _row(k):
                    for c in range(0, D, L):
                        cmp[k, pl.ds(c, L)] = buf[k, pl.ds(c, L)] * SCALE

                pltpu.sync_copy(cmp, out_hbm.at[pl.ds(base + cc * C, C)])

                @pl.when(cc + 2 < NCH)
                def _next():
                    issue(cc + 2, b)

    return emb


_emb = _make_sc_kernel()


def kernel(x, table):
    xf = x.astype(jnp.int32).reshape(-1)
    tab2 = jnp.pad(table, ((0, 0), (0, DP - D)))
    out = _emb(xf, tab2)
    return out.reshape(x.shape[0], x.shape[1], D)


# speedup vs baseline: 4.1831x; 1.3675x over previous
import functools
import math

import jax
import jax.numpy as jnp
from jax import lax
from jax.experimental import pallas as pl
from jax.experimental.pallas import tpu as pltpu
from jax.experimental.pallas import tpu_sc as plsc

VOCAB_SIZE = 1000000
D = 64
DP = 128
SCALE = math.sqrt(D)
NC, NS = 2, 16
NW = NC * NS
B = 4096 * 200
PER_W = B // NW
C = 128
NCH = PER_W // C
L = 16


VB = 16384


def _transpose_body(tt_ref, out_ref):
    out_ref[:, :D] = tt_ref[...].T


_transpose_pad = pl.pallas_call(
    _transpose_body,
    grid=((VOCAB_SIZE + VB - 1) // VB,),
    in_specs=[pl.BlockSpec((D, VB), lambda i: (0, i))],
    out_specs=pl.BlockSpec((VB, DP), lambda i: (i, 0)),
    out_shape=jax.ShapeDtypeStruct((VOCAB_SIZE, DP), jnp.float32),
)


def _make_sc_kernel():
    mesh = plsc.VectorSubcoreMesh(core_axis_name="c", subcore_axis_name="s")

    @functools.partial(
        pl.kernel,
        out_type=jax.ShapeDtypeStruct((B, D), jnp.float32),
        mesh=mesh,
        compiler_params=pltpu.CompilerParams(use_tc_tiling_on_sc=True),
        scratch_types=[
            pltpu.VMEM((PER_W,), jnp.int32),
            pltpu.VMEM((C, DP), jnp.float32),
            pltpu.VMEM((C, DP), jnp.float32),
            pltpu.VMEM((C, D), jnp.float32),
            pltpu.VMEM((C, D), jnp.float32),
            pltpu.SemaphoreType.DMA,
            pltpu.SemaphoreType.DMA,
        ],
    )
    def emb(x_hbm, tab_hbm, out_hbm, idx_v, buf0, buf1, cmp0, cmp1,
            sem0, sem1):
        wid = lax.axis_index("s") * NC + lax.axis_index("c")
        base = wid * PER_W
        pltpu.sync_copy(x_hbm.at[pl.ds(base, PER_W)], idx_v)
        bufs = (buf0, buf1)
        cmps = (cmp0, cmp1)
        sems = (sem0, sem1)

        def issue(ch, b):
            pltpu.async_copy(
                tab_hbm.at[idx_v.at[pl.ds(ch * C, C)]], bufs[b], sems[b])

        def wait_gather(ch, b):
            pltpu.make_async_copy(
                tab_hbm.at[idx_v.at[pl.ds(ch * C, C)]], bufs[b],
                sems[b]).wait()

        for b in range(2):
            issue(b, b)

        @pl.loop(0, NCH, step=2)
        def _chunk(ch):
            for b in range(2):
                cc = ch + b
                wait_gather(cc, b)
                buf, cmp = bufs[b], cmps[b]

                @pl.loop(0, C)
                def _row(k):
                    for c in range(0, D, L):
                        cmp[k, pl.ds(c, L)] = buf[k, pl.ds(c, L)] * SCALE

                pltpu.sync_copy(cmp, out_hbm.at[pl.ds(base + cc * C, C)])

                @pl.when(cc + 2 < NCH)
                def _next():
                    issue(cc + 2, b)

    return emb


_emb = _make_sc_kernel()


def kernel(x, table):
    xf = x.astype(jnp.int32).reshape(-1)
    tab2 = _transpose_pad(table.T)
    out = _emb(xf, tab2)
    return out.reshape(x.shape[0], x.shape[1], D)


# speedup vs baseline: 4.2797x; 1.0231x over previous
import functools
import math

import jax
import jax.numpy as jnp
from jax import lax
from jax.experimental import pallas as pl
from jax.experimental.pallas import tpu as pltpu
from jax.experimental.pallas import tpu_sc as plsc

VOCAB_SIZE = 1000000
D = 64
DP = 128
SCALE = math.sqrt(D)
NC, NS = 2, 16
NW = NC * NS
B = 4096 * 200
PER_W = B // NW
C = 160
NCH = PER_W // C
L = 16


VB = 16384


def _transpose_body(tt_ref, out_ref):
    out_ref[:, :D] = tt_ref[...].T


_transpose_pad = pl.pallas_call(
    _transpose_body,
    grid=((VOCAB_SIZE + VB - 1) // VB,),
    in_specs=[pl.BlockSpec((D, VB), lambda i: (0, i))],
    out_specs=pl.BlockSpec((VB, DP), lambda i: (i, 0)),
    out_shape=jax.ShapeDtypeStruct((VOCAB_SIZE, DP), jnp.float32),
)


def _make_sc_kernel():
    mesh = plsc.VectorSubcoreMesh(core_axis_name="c", subcore_axis_name="s")

    @functools.partial(
        pl.kernel,
        out_type=jax.ShapeDtypeStruct((B, D), jnp.float32),
        mesh=mesh,
        compiler_params=pltpu.CompilerParams(use_tc_tiling_on_sc=True),
        scratch_types=[
            pltpu.VMEM((PER_W,), jnp.int32),
            pltpu.VMEM((C, DP), jnp.float32),
            pltpu.VMEM((C, DP), jnp.float32),
            pltpu.VMEM((C, D), jnp.float32),
            pltpu.VMEM((C, D), jnp.float32),
            pltpu.SemaphoreType.DMA,
            pltpu.SemaphoreType.DMA,
            pltpu.SemaphoreType.DMA,
            pltpu.SemaphoreType.DMA,
        ],
    )
    def emb(x_hbm, tab_hbm, out_hbm, idx_v, buf0, buf1, cmp0, cmp1,
            sem0, sem1, osem0, osem1):
        wid = lax.axis_index("s") * NC + lax.axis_index("c")
        base = wid * PER_W
        pltpu.sync_copy(x_hbm.at[pl.ds(base, PER_W)], idx_v)
        bufs = (buf0, buf1)
        cmps = (cmp0, cmp1)
        sems = (sem0, sem1)
        osems = (osem0, osem1)

        def issue(ch, b):
            pltpu.async_copy(
                tab_hbm.at[idx_v.at[pl.ds(ch * C, C)]], bufs[b], sems[b])

        def wait_gather(ch, b):
            pltpu.make_async_copy(
                tab_hbm.at[idx_v.at[pl.ds(ch * C, C)]], bufs[b],
                sems[b]).wait()

        for b in range(2):
            issue(b, b)

        @pl.loop(0, NCH, step=2)
        def _chunk(ch):
            for b in range(2):
                cc = ch + b
                wait_gather(cc, b)
                buf, cmp = bufs[b], cmps[b]

                @pl.when(cc >= 2)
                def _drain():
                    pltpu.make_async_copy(
                        cmp, out_hbm.at[pl.ds(base + (cc - 2) * C, C)],
                        osems[b]).wait()

                @pl.loop(0, C)
                def _row(k):
                    for c in range(0, D, L):
                        cmp[k, pl.ds(c, L)] = buf[k, pl.ds(c, L)] * SCALE

                pltpu.async_copy(
                    cmp, out_hbm.at[pl.ds(base + cc * C, C)], osems[b])

                @pl.when(cc + 2 < NCH)
                def _next():
                    issue(cc + 2, b)

        for b in range(2):
            pltpu.make_async_copy(
                cmps[b],
                out_hbm.at[pl.ds(base + (NCH - 2 + b) * C, C)],
                osems[b]).wait()

    return emb


_emb = _make_sc_kernel()


def kernel(x, table):
    xf = x.astype(jnp.int32).reshape(-1)
    tab2 = _transpose_pad(table.T)
    out = _emb(xf, tab2)
    return out.reshape(x.shape[0], x.shape[1], D)


# speedup vs baseline: 4.3156x; 1.0084x over previous
import functools
import math

import jax
import jax.numpy as jnp
from jax import lax
from jax.experimental import pallas as pl
from jax.experimental.pallas import tpu as pltpu
from jax.experimental.pallas import tpu_sc as plsc

VOCAB_SIZE = 1000000
D = 64
DP = 128
SCALE = math.sqrt(D)
NC, NS = 2, 16
NW = NC * NS
B = 4096 * 200
PER_W = B // NW
C = 160
NCH = PER_W // C
L = 16


VB = 32768


def _transpose_body(tt_ref, out_ref):
    out_ref[:, :D] = tt_ref[...].T


_transpose_pad = pl.pallas_call(
    _transpose_body,
    grid=((VOCAB_SIZE + VB - 1) // VB,),
    in_specs=[pl.BlockSpec((D, VB), lambda i: (0, i))],
    out_specs=pl.BlockSpec((VB, DP), lambda i: (i, 0)),
    out_shape=jax.ShapeDtypeStruct((VOCAB_SIZE, DP), jnp.float32),
)


def _make_sc_kernel():
    mesh = plsc.VectorSubcoreMesh(core_axis_name="c", subcore_axis_name="s")

    @functools.partial(
        pl.kernel,
        out_type=jax.ShapeDtypeStruct((B, D), jnp.float32),
        mesh=mesh,
        compiler_params=pltpu.CompilerParams(use_tc_tiling_on_sc=True),
        scratch_types=[
            pltpu.VMEM((PER_W,), jnp.int32),
            pltpu.VMEM((C, DP), jnp.float32),
            pltpu.VMEM((C, DP), jnp.float32),
            pltpu.VMEM((C, D), jnp.float32),
            pltpu.VMEM((C, D), jnp.float32),
            pltpu.SemaphoreType.DMA,
            pltpu.SemaphoreType.DMA,
            pltpu.SemaphoreType.DMA,
            pltpu.SemaphoreType.DMA,
        ],
    )
    def emb(x_hbm, tab_hbm, out_hbm, idx_v, buf0, buf1, cmp0, cmp1,
            sem0, sem1, osem0, osem1):
        wid = lax.axis_index("s") * NC + lax.axis_index("c")
        base = wid * PER_W
        pltpu.sync_copy(x_hbm.at[pl.ds(base, PER_W)], idx_v)
        bufs = (buf0, buf1)
        cmps = (cmp0, cmp1)
        sems = (sem0, sem1)
        osems = (osem0, osem1)

        def issue(ch, b):
            pltpu.async_copy(
                tab_hbm.at[idx_v.at[pl.ds(ch * C, C)]], bufs[b], sems[b])

        def wait_gather(ch, b):
            pltpu.make_async_copy(
                tab_hbm.at[idx_v.at[pl.ds(ch * C, C)]], bufs[b],
                sems[b]).wait()

        for b in range(2):
            issue(b, b)

        @pl.loop(0, NCH, step=2)
        def _chunk(ch):
            for b in range(2):
                cc = ch + b
                wait_gather(cc, b)
                buf, cmp = bufs[b], cmps[b]

                @pl.when(cc >= 2)
                def _drain():
                    pltpu.make_async_copy(
                        cmp, out_hbm.at[pl.ds(base + (cc - 2) * C, C)],
                        osems[b]).wait()

                @pl.loop(0, C)
                def _row(k):
                    for c in range(0, D, L):
                        cmp[k, pl.ds(c, L)] = buf[k, pl.ds(c, L)] * SCALE

                pltpu.async_copy(
                    cmp, out_hbm.at[pl.ds(base + cc * C, C)], osems[b])

                @pl.when(cc + 2 < NCH)
                def _next():
                    issue(cc + 2, b)

        for b in range(2):
            pltpu.make_async_copy(
                cmps[b],
                out_hbm.at[pl.ds(base + (NCH - 2 + b) * C, C)],
                osems[b]).wait()

    return emb


_emb = _make_sc_kernel()


def kernel(x, table):
    xf = x.astype(jnp.int32).reshape(-1)
    tab2 = _transpose_pad(table.T)
    out = _emb(xf, tab2)
    return out.reshape(x.shape[0], x.shape[1], D)
